# baseline (speedup 1.0000x reference)
import jax
import jax.numpy as jnp
from jax import lax
from jax.experimental import pallas as pl
from jax.experimental.pallas import tpu as pltpu
from jax.experimental.pallas import tpu_sc as plsc

NUM_RAYS = 32768
NS = 64
NB = NS + 1
NOUT = 2 * NB
L = 16
NI = 4

_info = plsc.get_sparse_core_info()
NWORK = _info.num_cores * _info.num_subcores
RAYS_PER_W = NUM_RAYS // NWORK

CHUNK = 128
GPC = CHUNK // (L * NI)
NCHUNK = RAYS_PER_W // CHUNK


def _body(w_hbm, e_hbm, near_hbm, far_hbm, out_hbm,
          w_c, e_c, near_c, far_c, out_c, *cfhf):
    nc = _info.num_cores
    wid = lax.axis_index("s") * nc + lax.axis_index("c")
    craw = list(cfhf[:NI])
    cf = list(cfhf[NI:2 * NI])
    hf = list(cfhf[2 * NI:])
    lanes = lax.iota(jnp.int32, L)
    lanes_m = lanes - L
    ones_i = jnp.full((L,), 1, jnp.int32)
    zeros_i = jnp.zeros((L,), jnp.int32)
    zerosf = jnp.zeros((L,), jnp.float32)

    for s in range(NI):
        cf[s][pl.ds(0, L)] = zerosf
        hf[s][pl.ds(0, L)] = ones_i

    def hinit(j, carry):
        for s in range(NI):
            hf[s][pl.ds(j * L, L)] = zeros_i
        return carry

    lax.fori_loop(1, NB, hinit, 0)

    def chunk(cidx, carry):
        base = wid * RAYS_PER_W + cidx * CHUNK
        blk = wid * NCHUNK + cidx
        pltpu.sync_copy(w_hbm.at[blk], w_c.at[pl.ds(0, NS), :])
        pltpu.sync_copy(e_hbm.at[blk], e_c.at[pl.ds(0, NB), :])
        pltpu.sync_copy(near_hbm.at[pl.ds(base, CHUNK)], near_c)
        pltpu.sync_copy(far_hbm.at[pl.ds(base, CHUNK)], far_c)
        for t in range(CHUNK // L):
            e_c[NB, pl.ds(t * L, L)] = e_c[NS, pl.ds(t * L, L)]

        def quad(gq, carry2):
            offs = [gq * (L * NI) + s * L for s in range(NI)]
            rows = [offs[s] + lanes for s in range(NI)]
            near = [near_c[pl.ds(offs[s], L)] for s in range(NI)]
            fn = [far_c[pl.ds(offs[s], L)] - near[s] for s in range(NI)]

            def p1(i, cs):
                out = []
                for s in range(NI):
                    wi = w_c[i, pl.ds(offs[s], L)]
                    cn = cs[s] + wi
                    craw[s][pl.ds((i + 1) * L, L)] = cn
                    out.append(cn)
                return tuple(out)

            csum = plsc.parallel_loop(0, NS, unroll=4,
                                      carry=(zerosf,) * NI)(p1)
            k = []
            r = []
            for s in range(NI):
                ws = csum[s] + float(NS) * 0.01
                pad = jnp.maximum(1e-5 - ws, 0.0)
                k.append(pad * (1.0 / NS) + 0.01)
                r.append(1.0 / (ws + pad))

            for s in range(NI):
                e0 = e_c[0, pl.ds(offs[s], L)]
                plsc.store_scatter(out_c, [rows[s], zeros_i],
                                   e0 * fn[s] + near[s])

            def p2(i, kks):
                out = []
                for s in range(NI):
                    kk = kks[s] + k[s]
                    cr = craw[s][pl.ds(i * L, L)]
                    cdf = jnp.minimum(1.0, (cr + kk) * r[s])
                    cf[s][pl.ds(i * L, L)] = cdf
                    x = cdf * float(NB) - 0.5
                    ti = x.astype(jnp.int32)
                    p = ti + (x > ti.astype(jnp.float32)).astype(jnp.int32)
                    plsc.addupdate_scatter(hf[s], [p * L + lanes], ones_i)
                    ei = e_c[i, pl.ds(offs[s], L)]
                    plsc.store_scatter(out_c, [rows[s], p + i],
                                       ei * fn[s] + near[s])
                    out.append(kk)
                return tuple(out)

            plsc.parallel_loop(1, NB, unroll=4, carry=(zerosf,) * NI)(p2)
            for s in range(NI):
                cf[s][pl.ds(NB * L, L)] = cf[s][pl.ds(NS * L, L)]

            def p3(j, st):
                runs, us = st
                nruns = []
                nus = []
                for s in range(NI):
                    h = hf[s][pl.ds(j * L, L)]
                    run = runs[s] + h
                    u = us[s] + (1.0 / NB)
                    a0 = run * L + lanes_m
                    g0 = plsc.load_gather(cf[s], [a0])
                    g1 = plsc.load_gather(cf[s], [a0 + L])
                    bel = run - 1
                    b0 = plsc.load_gather(e_c, [bel, rows[s]])
                    b1 = plsc.load_gather(e_c, [run, rows[s]])
                    denom = g1 - g0
                    ok = denom > 1e-12
                    sd = jnp.where(ok, denom, 1.0)
                    t = jnp.where(ok, (u - g0) / sd, 0.0)
                    t = jnp.clip(t, 0.0, 1.0)
                    bins = b0 + t * (b1 - b0)
                    plsc.store_scatter(out_c, [rows[s], run + j],
                                       bins * fn[s] + near[s])
                    nruns.append(run)
                    nus.append(u)
                return (tuple(nruns), tuple(nus))

            u0 = jnp.full((L,), -0.5 / NB, jnp.float32)
            plsc.parallel_loop(0, NB, unroll=4,
                               carry=((zeros_i,) * NI, (u0,) * NI))(p3)

            def hclr(j):
                for s in range(NI):
                    hf[s][pl.ds(j * L, L)] = zeros_i

            plsc.parallel_loop(1, NB, unroll=4)(hclr)
            for s in range(NI):
                hf[s][pl.ds(0, L)] = ones_i
            return carry2

        lax.fori_loop(0, GPC, quad, 0)
        pltpu.sync_copy(out_c, out_hbm.at[pl.ds(base, CHUNK), :])
        return carry

    lax.fori_loop(0, NCHUNK, chunk, 0)


@jax.jit
def _run(w2, e2, n1, f1):
    mesh = plsc.VectorSubcoreMesh(core_axis_name="c", subcore_axis_name="s")
    fn = pl.kernel(
        _body,
        out_type=jax.ShapeDtypeStruct((NUM_RAYS, NOUT), jnp.float32),
        mesh=mesh,
        compiler_params=pltpu.CompilerParams(needs_layout_passes=False),
        scratch_types=[
            pltpu.VMEM((NS, CHUNK), jnp.float32),
            pltpu.VMEM((NB + 1, CHUNK), jnp.float32),
            pltpu.VMEM((CHUNK,), jnp.float32),
            pltpu.VMEM((CHUNK,), jnp.float32),
            pltpu.VMEM((CHUNK, NOUT), jnp.float32),
        ] + [pltpu.VMEM(((NB + 1) * L,), jnp.float32) for _ in range(2 * NI)]
          + [pltpu.VMEM(((NB + 1) * L,), jnp.int32) for _ in range(NI)],
    )
    return fn(w2, e2, n1, f1)


def kernel(weights, existing_bins, nears, fars):
    nblk = NWORK * NCHUNK
    w3 = weights[..., 0].T.reshape(NS, nblk, CHUNK).transpose(1, 0, 2)
    e3 = existing_bins.T.reshape(NB, nblk, CHUNK).transpose(1, 0, 2)
    return _run(w3, e3, nears[:, 0], fars[:, 0])


# speedup vs baseline: 1.4026x; 1.4026x over previous
import jax
import jax.numpy as jnp
from jax import lax
from jax.experimental import pallas as pl
from jax.experimental.pallas import tpu as pltpu
from jax.experimental.pallas import tpu_sc as plsc

NUM_RAYS = 32768
NS = 64
NB = NS + 1
NOUT = 2 * NB
L = 16
NI = 4

_info = plsc.get_sparse_core_info()
NWORK = _info.num_cores * _info.num_subcores
RAYS_PER_W = NUM_RAYS // NWORK

CHUNK = 128
GPC = CHUNK // (L * NI)
NCHUNK = RAYS_PER_W // CHUNK


def _body(w_hbm, e_hbm, near_hbm, far_hbm, out_hbm,
          w_c, e_c, near_c, far_c, out_c, *cfhf):
    nc = _info.num_cores
    wid = lax.axis_index("s") * nc + lax.axis_index("c")
    craw = list(cfhf[:NI])
    cf = list(cfhf[NI:2 * NI])
    hf = list(cfhf[2 * NI:])
    lanes = lax.iota(jnp.int32, L)
    lanes_m = lanes - L
    ones_i = jnp.full((L,), 1, jnp.int32)
    zeros_i = jnp.zeros((L,), jnp.int32)
    zerosf = jnp.zeros((L,), jnp.float32)

    for s in range(NI):
        cf[s][pl.ds(0, L)] = zerosf
        hf[s][pl.ds(0, L)] = ones_i

    def hinit(j, carry):
        for s in range(NI):
            hf[s][pl.ds(j * L, L)] = zeros_i
        return carry

    lax.fori_loop(1, NB, hinit, 0)

    def chunk(cidx, carry):
        base = wid * RAYS_PER_W + cidx * CHUNK
        blk = wid * NCHUNK + cidx
        pltpu.sync_copy(w_hbm.at[blk], w_c.at[pl.ds(0, NS), :])
        pltpu.sync_copy(e_hbm.at[blk], e_c.at[pl.ds(0, NB), :])
        pltpu.sync_copy(near_hbm.at[pl.ds(base, CHUNK)], near_c)
        pltpu.sync_copy(far_hbm.at[pl.ds(base, CHUNK)], far_c)
        for t in range(CHUNK // L):
            e_c[NB, pl.ds(t * L, L)] = e_c[NS, pl.ds(t * L, L)]

        def quad(gq, carry2):
            offs = [gq * (L * NI) + s * L for s in range(NI)]
            rows = [offs[s] + lanes for s in range(NI)]
            near = [near_c[pl.ds(offs[s], L)] for s in range(NI)]
            fn = [far_c[pl.ds(offs[s], L)] - near[s] for s in range(NI)]

            def p1(i, cs):
                out = []
                for s in range(NI):
                    wi = w_c[i, pl.ds(offs[s], L)]
                    cn = cs[s] + wi
                    craw[s][pl.ds((i + 1) * L, L)] = cn
                    out.append(cn)
                return tuple(out)

            csum = plsc.parallel_loop(0, NS, unroll=2,
                                      carry=(zerosf,) * NI)(p1)
            k = []
            r = []
            for s in range(NI):
                ws = csum[s] + float(NS) * 0.01
                pad = jnp.maximum(1e-5 - ws, 0.0)
                k.append(pad * (1.0 / NS) + 0.01)
                r.append(1.0 / (ws + pad))

            for s in range(NI):
                e0 = e_c[0, pl.ds(offs[s], L)]
                plsc.store_scatter(out_c, [rows[s], zeros_i],
                                   e0 * fn[s] + near[s])

            def p2(i, kks):
                out = []
                for s in range(NI):
                    kk = kks[s] + k[s]
                    cr = craw[s][pl.ds(i * L, L)]
                    cdf = jnp.minimum(1.0, (cr + kk) * r[s])
                    cf[s][pl.ds(i * L, L)] = cdf
                    x = cdf * float(NB) - 0.5
                    ti = x.astype(jnp.int32)
                    p = ti + (x > ti.astype(jnp.float32)).astype(jnp.int32)
                    plsc.addupdate_scatter(hf[s], [p * L + lanes], ones_i)
                    ei = e_c[i, pl.ds(offs[s], L)]
                    plsc.store_scatter(out_c, [rows[s], p + i],
                                       ei * fn[s] + near[s])
                    out.append(kk)
                return tuple(out)

            plsc.parallel_loop(1, NB, unroll=2, carry=(zerosf,) * NI)(p2)
            for s in range(NI):
                cf[s][pl.ds(NB * L, L)] = cf[s][pl.ds(NS * L, L)]

            def p3(j, st):
                runs, us = st
                nruns = []
                nus = []
                for s in range(NI):
                    h = hf[s][pl.ds(j * L, L)]
                    run = runs[s] + h
                    u = us[s] + (1.0 / NB)
                    a0 = run * L + lanes_m
                    g0 = plsc.load_gather(cf[s], [a0])
                    g1 = plsc.load_gather(cf[s], [a0 + L])
                    bel = run - 1
                    b0 = plsc.load_gather(e_c, [bel, rows[s]])
                    b1 = plsc.load_gather(e_c, [run, rows[s]])
                    denom = g1 - g0
                    ok = denom > 1e-12
                    sd = jnp.where(ok, denom, 1.0)
                    t = jnp.where(ok, (u - g0) / sd, 0.0)
                    t = jnp.clip(t, 0.0, 1.0)
                    bins = b0 + t * (b1 - b0)
                    plsc.store_scatter(out_c, [rows[s], run + j],
                                       bins * fn[s] + near[s])
                    nruns.append(run)
                    nus.append(u)
                return (tuple(nruns), tuple(nus))

            u0 = jnp.full((L,), -0.5 / NB, jnp.float32)
            plsc.parallel_loop(0, NB, unroll=2,
                               carry=((zeros_i,) * NI, (u0,) * NI))(p3)

            def hclr(j):
                for s in range(NI):
                    hf[s][pl.ds(j * L, L)] = zeros_i

            plsc.parallel_loop(1, NB, unroll=4)(hclr)
            for s in range(NI):
                hf[s][pl.ds(0, L)] = ones_i
            return carry2

        lax.fori_loop(0, GPC, quad, 0)
        pltpu.sync_copy(out_c, out_hbm.at[pl.ds(base, CHUNK), :])
        return carry

    lax.fori_loop(0, NCHUNK, chunk, 0)


@jax.jit
def _run(w2, e2, n1, f1):
    mesh = plsc.VectorSubcoreMesh(core_axis_name="c", subcore_axis_name="s")
    fn = pl.kernel(
        _body,
        out_type=jax.ShapeDtypeStruct((NUM_RAYS, NOUT), jnp.float32),
        mesh=mesh,
        compiler_params=pltpu.CompilerParams(needs_layout_passes=False),
        scratch_types=[
            pltpu.VMEM((NS, CHUNK), jnp.float32),
            pltpu.VMEM((NB + 1, CHUNK), jnp.float32),
            pltpu.VMEM((CHUNK,), jnp.float32),
            pltpu.VMEM((CHUNK,), jnp.float32),
            pltpu.VMEM((CHUNK, NOUT), jnp.float32),
        ] + [pltpu.VMEM(((NB + 1) * L,), jnp.float32) for _ in range(2 * NI)]
          + [pltpu.VMEM(((NB + 1) * L,), jnp.int32) for _ in range(NI)],
    )
    return fn(w2, e2, n1, f1)


def kernel(weights, existing_bins, nears, fars):
    nblk = NWORK * NCHUNK
    w3 = weights[..., 0].T.reshape(NS, nblk, CHUNK).transpose(1, 0, 2)
    e3 = existing_bins.T.reshape(NB, nblk, CHUNK).transpose(1, 0, 2)
    return _run(w3, e3, nears[:, 0], fars[:, 0])


# speedup vs baseline: 1.5644x; 1.1154x over previous
import jax
import jax.numpy as jnp
from jax import lax
from jax.experimental import pallas as pl
from jax.experimental.pallas import tpu as pltpu
from jax.experimental.pallas import tpu_sc as plsc

NUM_RAYS = 32768
NS = 64
NB = NS + 1
NOUT = 2 * NB
L = 16
NI = 4

_info = plsc.get_sparse_core_info()
NWORK = _info.num_cores * _info.num_subcores
RAYS_PER_W = NUM_RAYS // NWORK

CHUNK = 128
GPC = CHUNK // (L * NI)
NCHUNK = RAYS_PER_W // CHUNK


def _body(w_hbm, e_hbm, near_hbm, far_hbm, out_hbm,
          w_c, e_c, near_c, far_c, out_c, *cfhf):
    nc = _info.num_cores
    wid = lax.axis_index("s") * nc + lax.axis_index("c")
    craw = list(cfhf[:NI])
    cf = list(cfhf[NI:2 * NI])
    hf = list(cfhf[2 * NI:3 * NI])
    sems = list(cfhf[3 * NI:])
    lanes = lax.iota(jnp.int32, L)
    lanes_m = lanes - L
    ones_i = jnp.full((L,), 1, jnp.int32)
    zeros_i = jnp.zeros((L,), jnp.int32)
    zerosf = jnp.zeros((L,), jnp.float32)

    for s in range(NI):
        cf[s][pl.ds(0, L)] = zerosf
        hf[s][pl.ds(0, L)] = ones_i

    def hinit(j, carry):
        for s in range(NI):
            hf[s][pl.ds(j * L, L)] = zeros_i
        return carry

    lax.fori_loop(1, NB, hinit, 0)

    def start_in(cidx, par):
        base = wid * RAYS_PER_W + cidx * CHUNK
        blk = wid * NCHUNK + cidx
        sem = sems[par]
        pltpu.async_copy(w_hbm.at[blk], w_c.at[pl.ds(par * NS, NS), :], sem)
        pltpu.async_copy(e_hbm.at[blk],
                         e_c.at[pl.ds(par * (NB + 1), NB), :], sem)
        pltpu.async_copy(near_hbm.at[pl.ds(base, CHUNK)],
                         near_c.at[pl.ds(par * CHUNK, CHUNK)], sem)
        pltpu.async_copy(far_hbm.at[pl.ds(base, CHUNK)],
                         far_c.at[pl.ds(par * CHUNK, CHUNK)], sem)

    def wait_in(cidx, par):
        base = wid * RAYS_PER_W + cidx * CHUNK
        blk = wid * NCHUNK + cidx
        sem = sems[par]
        pltpu.make_async_copy(w_hbm.at[blk],
                              w_c.at[pl.ds(par * NS, NS), :], sem).wait()
        pltpu.make_async_copy(e_hbm.at[blk],
                              e_c.at[pl.ds(par * (NB + 1), NB), :], sem).wait()
        pltpu.make_async_copy(near_hbm.at[pl.ds(base, CHUNK)],
                              near_c.at[pl.ds(par * CHUNK, CHUNK)], sem).wait()
        pltpu.make_async_copy(far_hbm.at[pl.ds(base, CHUNK)],
                              far_c.at[pl.ds(par * CHUNK, CHUNK)], sem).wait()

    def do_chunk(cidx, par):
        base = wid * RAYS_PER_W + cidx * CHUNK
        wb = par * NS
        eb = par * (NB + 1)
        nb_ = par * CHUNK
        for t in range(CHUNK // L):
            e_c[eb + NB, pl.ds(t * L, L)] = e_c[eb + NS, pl.ds(t * L, L)]

        def quad(gq, carry2):
            offs = [gq * (L * NI) + s * L for s in range(NI)]
            rows = [offs[s] + lanes for s in range(NI)]
            near = [near_c[pl.ds(nb_ + offs[s], L)] for s in range(NI)]
            fn = [far_c[pl.ds(nb_ + offs[s], L)] - near[s] for s in range(NI)]

            def p1(i, cs):
                out = []
                for s in range(NI):
                    wi = w_c[wb + i, pl.ds(offs[s], L)]
                    cn = cs[s] + wi
                    craw[s][pl.ds((i + 1) * L, L)] = cn
                    out.append(cn)
                return tuple(out)

            csum = plsc.parallel_loop(0, NS, unroll=2,
                                      carry=(zerosf,) * NI)(p1)
            k = []
            r = []
            for s in range(NI):
                ws = csum[s] + float(NS) * 0.01
                pad = jnp.maximum(1e-5 - ws, 0.0)
                k.append(pad * (1.0 / NS) + 0.01)
                r.append(1.0 / (ws + pad))

            for s in range(NI):
                e0 = e_c[eb, pl.ds(offs[s], L)]
                plsc.store_scatter(out_c, [rows[s], zeros_i],
                                   e0 * fn[s] + near[s])

            def p2(i, kks):
                out = []
                for s in range(NI):
                    kk = kks[s] + k[s]
                    cr = craw[s][pl.ds(i * L, L)]
                    cdf = jnp.minimum(1.0, (cr + kk) * r[s])
                    cf[s][pl.ds(i * L, L)] = cdf
                    x = cdf * float(NB) - 0.5
                    ti = x.astype(jnp.int32)
                    p = ti + (x > ti.astype(jnp.float32)).astype(jnp.int32)
                    plsc.addupdate_scatter(hf[s], [p * L + lanes], ones_i)
                    ei = e_c[eb + i, pl.ds(offs[s], L)]
                    plsc.store_scatter(out_c, [rows[s], p + i],
                                       ei * fn[s] + near[s])
                    out.append(kk)
                return tuple(out)

            plsc.parallel_loop(1, NB, unroll=2, carry=(zerosf,) * NI)(p2)
            for s in range(NI):
                cf[s][pl.ds(NB * L, L)] = cf[s][pl.ds(NS * L, L)]

            def p3(j, st):
                runs, us = st
                nruns = []
                nus = []
                for s in range(NI):
                    h = hf[s][pl.ds(j * L, L)]
                    run = runs[s] + h
                    u = us[s] + (1.0 / NB)
                    a0 = run * L + lanes_m
                    g0 = plsc.load_gather(cf[s], [a0])
                    g1 = plsc.load_gather(cf[s], [a0 + L])
                    bel = run + (eb - 1)
                    b0 = plsc.load_gather(e_c, [bel, rows[s]])
                    b1 = plsc.load_gather(e_c, [bel + 1, rows[s]])
                    denom = g1 - g0
                    ok = denom > 1e-12
                    sd = jnp.where(ok, denom, 1.0)
                    t = jnp.where(ok, (u - g0) / sd, 0.0)
                    t = jnp.clip(t, 0.0, 1.0)
                    bins = b0 + t * (b1 - b0)
                    plsc.store_scatter(out_c, [rows[s], run + j],
                                       bins * fn[s] + near[s])
                    nruns.append(run)
                    nus.append(u)
                return (tuple(nruns), tuple(nus))

            u0 = jnp.full((L,), -0.5 / NB, jnp.float32)
            plsc.parallel_loop(0, NB, unroll=2,
                               carry=((zeros_i,) * NI, (u0,) * NI))(p3)

            def hclr(j):
                for s in range(NI):
                    hf[s][pl.ds(j * L, L)] = zeros_i

            plsc.parallel_loop(1, NB, unroll=4)(hclr)
            for s in range(NI):
                hf[s][pl.ds(0, L)] = ones_i
            return carry2

        lax.fori_loop(0, GPC, quad, 0)
        pltpu.sync_copy(out_c, out_hbm.at[pl.ds(base, CHUNK), :])

    start_in(0, 0)

    def chunk2(c2, carry):
        for par in range(2):
            cidx = c2 * 2 + par

            @pl.when(cidx < NCHUNK - 1)
            def _():
                start_in(cidx + 1, 1 - par)

            wait_in(cidx, par)
            do_chunk(cidx, par)
        return carry

    lax.fori_loop(0, NCHUNK // 2, chunk2, 0)


@jax.jit
def _run(w2, e2, n1, f1):
    mesh = plsc.VectorSubcoreMesh(core_axis_name="c", subcore_axis_name="s")
    fn = pl.kernel(
        _body,
        out_type=jax.ShapeDtypeStruct((NUM_RAYS, NOUT), jnp.float32),
        mesh=mesh,
        compiler_params=pltpu.CompilerParams(needs_layout_passes=False),
        scratch_types=[
            pltpu.VMEM((2 * NS, CHUNK), jnp.float32),
            pltpu.VMEM((2 * (NB + 1), CHUNK), jnp.float32),
            pltpu.VMEM((2 * CHUNK,), jnp.float32),
            pltpu.VMEM((2 * CHUNK,), jnp.float32),
            pltpu.VMEM((CHUNK, NOUT), jnp.float32),
        ] + [pltpu.VMEM(((NB + 1) * L,), jnp.float32) for _ in range(2 * NI)]
          + [pltpu.VMEM(((NB + 1) * L,), jnp.int32) for _ in range(NI)]
          + [pltpu.SemaphoreType.DMA for _ in range(2)],
    )
    return fn(w2, e2, n1, f1)


def kernel(weights, existing_bins, nears, fars):
    nblk = NWORK * NCHUNK
    w3 = weights[..., 0].T.reshape(NS, nblk, CHUNK).transpose(1, 0, 2)
    e3 = existing_bins.T.reshape(NB, nblk, CHUNK).transpose(1, 0, 2)
    return _run(w3, e3, nears[:, 0], fars[:, 0])


# speedup vs baseline: 1.6286x; 1.0410x over previous
import jax
import jax.numpy as jnp
from jax import lax
from jax.experimental import pallas as pl
from jax.experimental.pallas import tpu as pltpu
from jax.experimental.pallas import tpu_sc as plsc

NUM_RAYS = 32768
NS = 64
NB = NS + 1
NOUT = 2 * NB
L = 16
NI = 2

_info = plsc.get_sparse_core_info()
NWORK = _info.num_cores * _info.num_subcores
RAYS_PER_W = NUM_RAYS // NWORK

CHUNK = 128
GPC = CHUNK // (L * NI)
NCHUNK = RAYS_PER_W // CHUNK


def _body(w_hbm, e_hbm, near_hbm, far_hbm, out_hbm,
          w_c, e_c, near_c, far_c, out_c, *cfhf):
    nc = _info.num_cores
    wid = lax.axis_index("s") * nc + lax.axis_index("c")
    craw = list(cfhf[:NI])
    cf = list(cfhf[NI:2 * NI])
    hf = list(cfhf[2 * NI:3 * NI])
    sems = list(cfhf[3 * NI:])
    lanes = lax.iota(jnp.int32, L)
    lanes_m = lanes - L
    ones_i = jnp.full((L,), 1, jnp.int32)
    zeros_i = jnp.zeros((L,), jnp.int32)
    zerosf = jnp.zeros((L,), jnp.float32)

    for s in range(NI):
        cf[s][pl.ds(0, L)] = zerosf
        hf[s][pl.ds(0, L)] = ones_i

    def hinit(j, carry):
        for s in range(NI):
            hf[s][pl.ds(j * L, L)] = zeros_i
        return carry

    lax.fori_loop(1, NB, hinit, 0)

    def start_in(cidx, par):
        base = wid * RAYS_PER_W + cidx * CHUNK
        blk = wid * NCHUNK + cidx
        sem = sems[par]
        pltpu.async_copy(w_hbm.at[blk], w_c.at[pl.ds(par * NS, NS), :], sem)
        pltpu.async_copy(e_hbm.at[blk],
                         e_c.at[pl.ds(par * (NB + 1), NB), :], sem)
        pltpu.async_copy(near_hbm.at[pl.ds(base, CHUNK)],
                         near_c.at[pl.ds(par * CHUNK, CHUNK)], sem)
        pltpu.async_copy(far_hbm.at[pl.ds(base, CHUNK)],
                         far_c.at[pl.ds(par * CHUNK, CHUNK)], sem)

    def wait_in(cidx, par):
        base = wid * RAYS_PER_W + cidx * CHUNK
        blk = wid * NCHUNK + cidx
        sem = sems[par]
        pltpu.make_async_copy(w_hbm.at[blk],
                              w_c.at[pl.ds(par * NS, NS), :], sem).wait()
        pltpu.make_async_copy(e_hbm.at[blk],
                              e_c.at[pl.ds(par * (NB + 1), NB), :], sem).wait()
        pltpu.make_async_copy(near_hbm.at[pl.ds(base, CHUNK)],
                              near_c.at[pl.ds(par * CHUNK, CHUNK)], sem).wait()
        pltpu.make_async_copy(far_hbm.at[pl.ds(base, CHUNK)],
                              far_c.at[pl.ds(par * CHUNK, CHUNK)], sem).wait()

    def do_chunk(cidx, par):
        base = wid * RAYS_PER_W + cidx * CHUNK
        wb = par * NS
        eb = par * (NB + 1)
        nb_ = par * CHUNK
        for t in range(CHUNK // L):
            e_c[eb + NB, pl.ds(t * L, L)] = e_c[eb + NS, pl.ds(t * L, L)]

        def quad(gq, carry2):
            offs = [gq * (L * NI) + s * L for s in range(NI)]
            rows = [offs[s] + lanes for s in range(NI)]
            near = [near_c[pl.ds(nb_ + offs[s], L)] for s in range(NI)]
            fn = [far_c[pl.ds(nb_ + offs[s], L)] - near[s] for s in range(NI)]

            def p1(i, cs):
                out = []
                for s in range(NI):
                    wi = w_c[wb + i, pl.ds(offs[s], L)]
                    cn = cs[s] + wi
                    craw[s][pl.ds((i + 1) * L, L)] = cn
                    out.append(cn)
                return tuple(out)

            csum = plsc.parallel_loop(0, NS, unroll=4,
                                      carry=(zerosf,) * NI)(p1)
            k = []
            r = []
            for s in range(NI):
                ws = csum[s] + float(NS) * 0.01
                pad = jnp.maximum(1e-5 - ws, 0.0)
                k.append(pad * (1.0 / NS) + 0.01)
                r.append(1.0 / (ws + pad))

            for s in range(NI):
                e0 = e_c[eb, pl.ds(offs[s], L)]
                plsc.store_scatter(out_c, [rows[s], zeros_i],
                                   e0 * fn[s] + near[s])

            def p2(i, kks):
                out = []
                for s in range(NI):
                    kk = kks[s] + k[s]
                    cr = craw[s][pl.ds(i * L, L)]
                    cdf = jnp.minimum(1.0, (cr + kk) * r[s])
                    cf[s][pl.ds(i * L, L)] = cdf
                    x = cdf * float(NB) - 0.5
                    ti = x.astype(jnp.int32)
                    p = ti + (x > ti.astype(jnp.float32)).astype(jnp.int32)
                    plsc.addupdate_scatter(hf[s], [p * L + lanes], ones_i)
                    ei = e_c[eb + i, pl.ds(offs[s], L)]
                    plsc.store_scatter(out_c, [rows[s], p + i],
                                       ei * fn[s] + near[s])
                    out.append(kk)
                return tuple(out)

            plsc.parallel_loop(1, NB, unroll=4, carry=(zerosf,) * NI)(p2)
            for s in range(NI):
                cf[s][pl.ds(NB * L, L)] = cf[s][pl.ds(NS * L, L)]

            def p3(j, st):
                runs, us = st
                nruns = []
                nus = []
                for s in range(NI):
                    h = hf[s][pl.ds(j * L, L)]
                    run = runs[s] + h
                    u = us[s] + (1.0 / NB)
                    a0 = run * L + lanes_m
                    g0 = plsc.load_gather(cf[s], [a0])
                    g1 = plsc.load_gather(cf[s], [a0 + L])
                    bel = run + (eb - 1)
                    b0 = plsc.load_gather(e_c, [bel, rows[s]])
                    b1 = plsc.load_gather(e_c, [bel + 1, rows[s]])
                    denom = g1 - g0
                    ok = denom > 1e-12
                    sd = jnp.where(ok, denom, 1.0)
                    t = jnp.where(ok, (u - g0) / sd, 0.0)
                    t = jnp.clip(t, 0.0, 1.0)
                    bins = b0 + t * (b1 - b0)
                    plsc.store_scatter(out_c, [rows[s], run + j],
                                       bins * fn[s] + near[s])
                    nruns.append(run)
                    nus.append(u)
                return (tuple(nruns), tuple(nus))

            u0 = jnp.full((L,), -0.5 / NB, jnp.float32)
            plsc.parallel_loop(0, NB, unroll=4,
                               carry=((zeros_i,) * NI, (u0,) * NI))(p3)

            def hclr(j):
                for s in range(NI):
                    hf[s][pl.ds(j * L, L)] = zeros_i

            plsc.parallel_loop(1, NB, unroll=4)(hclr)
            for s in range(NI):
                hf[s][pl.ds(0, L)] = ones_i
            return carry2

        lax.fori_loop(0, GPC, quad, 0)
        pltpu.sync_copy(out_c, out_hbm.at[pl.ds(base, CHUNK), :])

    start_in(0, 0)

    def chunk2(c2, carry):
        for par in range(2):
            cidx = c2 * 2 + par

            @pl.when(cidx < NCHUNK - 1)
            def _():
                start_in(cidx + 1, 1 - par)

            wait_in(cidx, par)
            do_chunk(cidx, par)
        return carry

    lax.fori_loop(0, NCHUNK // 2, chunk2, 0)


@jax.jit
def _run(w2, e2, n1, f1):
    mesh = plsc.VectorSubcoreMesh(core_axis_name="c", subcore_axis_name="s")
    fn = pl.kernel(
        _body,
        out_type=jax.ShapeDtypeStruct((NUM_RAYS, NOUT), jnp.float32),
        mesh=mesh,
        compiler_params=pltpu.CompilerParams(needs_layout_passes=False),
        scratch_types=[
            pltpu.VMEM((2 * NS, CHUNK), jnp.float32),
            pltpu.VMEM((2 * (NB + 1), CHUNK), jnp.float32),
            pltpu.VMEM((2 * CHUNK,), jnp.float32),
            pltpu.VMEM((2 * CHUNK,), jnp.float32),
            pltpu.VMEM((CHUNK, NOUT), jnp.float32),
        ] + [pltpu.VMEM(((NB + 1) * L,), jnp.float32) for _ in range(2 * NI)]
          + [pltpu.VMEM(((NB + 1) * L,), jnp.int32) for _ in range(NI)]
          + [pltpu.SemaphoreType.DMA for _ in range(2)],
    )
    return fn(w2, e2, n1, f1)


def kernel(weights, existing_bins, nears, fars):
    nblk = NWORK * NCHUNK
    w3 = weights[..., 0].T.reshape(NS, nblk, CHUNK).transpose(1, 0, 2)
    e3 = existing_bins.T.reshape(NB, nblk, CHUNK).transpose(1, 0, 2)
    return _run(w3, e3, nears[:, 0], fars[:, 0])
